# Initial kernel scaffold; baseline (speedup 1.0000x reference)
import jax
import jax.numpy as jnp
from jax.experimental import pallas as pl


def kernel(x):
    raise NotImplementedError("write your pallas kernel here")


# speedup vs baseline: 1.8290x; 1.8290x over previous
import functools

import jax
import jax.numpy as jnp
from jax.experimental import pallas as pl

EPS_PD = 1e-6
MARGIN = 0.1


def _loss_kernel(x_ref, out_ref):
    a_raw = x_ref[:, 0, :]
    p_raw = x_ref[:, 1, :]
    B = a_raw.shape[0]
    D = a_raw.shape[1]

    na = jnp.maximum(
        jnp.sqrt(jnp.sum(a_raw * a_raw, axis=1, keepdims=True)), 1e-12)
    np_ = jnp.maximum(
        jnp.sqrt(jnp.sum(p_raw * p_raw, axis=1, keepdims=True)), 1e-12)
    a = a_raw / na
    p = p_raw / np_

    sa = jnp.sum(a * a, axis=1)
    sp = jnp.sum(p * p, axis=1)
    ra = jnp.sum(a, axis=1)
    rp = jnp.sum(p, axis=1)

    dot = jax.lax.dot_general(
        a, p, (((1,), (1,)), ((), ())), preferred_element_type=jnp.float32)

    sq = (sa[:, None] + sp[None, :] - 2.0 * dot
          + (2.0 * EPS_PD) * (ra[:, None] - rp[None, :])
          + D * EPS_PD * EPS_PD)

    rows = jax.lax.broadcasted_iota(jnp.int32, (B, B), 0)
    cols = jax.lax.broadcasted_iota(jnp.int32, (B, B), 1)
    sq = jnp.where(rows == cols, jnp.float32(1e18), sq)

    neg2 = jnp.maximum(jnp.min(sq, axis=1), 0.0)

    diff = a - p + EPS_PD
    pos2 = jnp.sum(diff * diff, axis=1)

    loss = jnp.mean(jax.nn.relu(pos2 - neg2 + MARGIN))
    out_ref[...] = loss.reshape(1, 1)


@functools.partial(jax.jit)
def kernel(x):
    out = pl.pallas_call(
        _loss_kernel,
        out_shape=jax.ShapeDtypeStruct((1, 1), jnp.float32),
    )(x)
    return out[0, 0]


# speedup vs baseline: 2.5980x; 1.4205x over previous
import functools

import jax
import jax.numpy as jnp
from jax.experimental import pallas as pl

EPS_PD = 1e-6
MARGIN = 0.1


def _loss_kernel(x_ref, out_ref):
    a_raw = x_ref[:, 0, :]
    p_raw = x_ref[:, 1, :]
    B = a_raw.shape[0]
    D = a_raw.shape[1]

    ones = jnp.ones((D, 128), dtype=jnp.float32)
    ssa = jax.lax.dot_general(
        a_raw * a_raw, ones, (((1,), (0,)), ((), ())),
        preferred_element_type=jnp.float32)
    ssp = jax.lax.dot_general(
        p_raw * p_raw, ones, (((1,), (0,)), ((), ())),
        preferred_element_type=jnp.float32)
    inva = jax.lax.rsqrt(jnp.maximum(ssa, 1e-24))
    invp = jax.lax.rsqrt(jnp.maximum(ssp, 1e-24))
    reps = D // 128
    inva = jnp.concatenate([inva] * reps, axis=1)
    invp = jnp.concatenate([invp] * reps, axis=1)
    a = a_raw * inva
    p = p_raw * invp

    dot = jax.lax.dot_general(
        a, p, (((1,), (1,)), ((), ())), preferred_element_type=jnp.float32)

    rows = jax.lax.broadcasted_iota(jnp.int32, (B, B), 0)
    cols = jax.lax.broadcasted_iota(jnp.int32, (B, B), 1)
    masked = jnp.where(rows == cols, jnp.float32(-2.0), dot)
    maxdot = jnp.max(masked, axis=1)
    neg2 = jnp.maximum(2.0 - 2.0 * maxdot, 0.0)

    diff = a - p + EPS_PD
    pos2 = jnp.sum(diff * diff, axis=1)

    loss = jnp.mean(jax.nn.relu(pos2 - neg2 + MARGIN))
    out_ref[...] = loss.reshape(1, 1)


@functools.partial(jax.jit)
def kernel(x):
    out = pl.pallas_call(
        _loss_kernel,
        out_shape=jax.ShapeDtypeStruct((1, 1), jnp.float32),
    )(x)
    return out[0, 0]


# speedup vs baseline: 2.8959x; 1.1147x over previous
import functools

import jax
import jax.numpy as jnp
from jax.experimental import pallas as pl

EPS_PD = 1e-6
MARGIN = 0.1


def _loss_kernel(x_ref, out_ref):
    a_raw = x_ref[:, 0, :]
    p_raw = x_ref[:, 1, :]
    B = a_raw.shape[0]
    D = a_raw.shape[1]

    ones = jnp.ones((D, 128), dtype=jnp.float32)
    dims = (((1,), (0,)), ((), ()))
    ssp = jax.lax.dot_general(
        p_raw * p_raw, ones, dims, preferred_element_type=jnp.float32)
    invp = jax.lax.rsqrt(jnp.maximum(ssp, 1e-24))
    invp = jnp.concatenate([invp] * (D // 128), axis=1)
    p = p_raw * invp

    ssa = jax.lax.dot_general(
        a_raw * a_raw, ones, dims, preferred_element_type=jnp.float32)
    inva = jax.lax.rsqrt(jnp.maximum(ssa[:, 0:1], 1e-24))

    gdiag = jax.lax.dot_general(
        a_raw * p, ones, dims, preferred_element_type=jnp.float32)

    g = jax.lax.dot_general(
        a_raw, p, (((1,), (1,)), ((), ())),
        preferred_element_type=jnp.float32)

    rows = jax.lax.broadcasted_iota(jnp.int32, (B, B), 0)
    cols = jax.lax.broadcasted_iota(jnp.int32, (B, B), 1)
    masked = jnp.where(rows == cols, jnp.float32(-1e30), g)
    rowmax = jnp.max(masked, axis=1, keepdims=True)

    neg2 = jnp.maximum(2.0 - 2.0 * (inva * rowmax), 0.0)
    pos2 = 2.0 - 2.0 * (inva * gdiag[:, 0:1])

    loss = jnp.mean(jax.nn.relu(pos2 - neg2 + MARGIN))
    out_ref[...] = loss.reshape(1, 1)


@functools.partial(jax.jit)
def kernel(x):
    out = pl.pallas_call(
        _loss_kernel,
        out_shape=jax.ShapeDtypeStruct((1, 1), jnp.float32),
    )(x)
    return out[0, 0]
